# baseline (speedup 1.0000x reference)
import functools

import jax
import jax.numpy as jnp
from jax import lax
from jax.experimental import pallas as pl
from jax.experimental.pallas import tpu as pltpu
from jax.experimental.pallas import tpu_sc as plsc

D_MODEL = 64
SCALE = 8.0

NC = 2
NS = 16
NW = NC * NS
TB = 128
NBUF = 4


@functools.lru_cache(maxsize=None)
def _emb_call(S: int, P: int):
    assert S % (NW * TB) == 0 or S == NW * TB
    mesh = plsc.VectorSubcoreMesh(core_axis_name="c", subcore_axis_name="s")
    n_rounds = P // NBUF
    block_words = (D_MODEL // 8) * TB

    scratch = (
        [pltpu.VMEM((P, TB), jnp.int32)]
        + [pltpu.VMEM((TB, D_MODEL), jnp.float32) for _ in range(NBUF)]
        + [pltpu.VMEM((D_MODEL * TB,), jnp.float32) for _ in range(NBUF)]
        + [pltpu.SemaphoreType.DMA for _ in range(2 * NBUF)]
    )

    @functools.partial(
        pl.kernel,
        mesh=mesh,
        out_type=jax.ShapeDtypeStruct((S * P * D_MODEL,), jnp.float32),
        scratch_types=scratch,
        compiler_params=pltpu.CompilerParams(
            use_tc_tiling_on_sc=False, needs_layout_passes=False),
    )
    def emb(xt_hbm, table_hbm, out_hbm, idx_v, *rest):
        gbuf = rest[:NBUF]
        tbuf = rest[NBUF:2 * NBUF]
        gsem = rest[2 * NBUF:3 * NBUF]
        ssem = rest[3 * NBUF:4 * NBUF]

        wid = lax.axis_index("s") * NC + lax.axis_index("c")
        pltpu.sync_copy(xt_hbm.at[pl.ds(0, P), pl.ds(wid * TB, TB)], idx_v)

        lane = lax.iota(jnp.int32, 16)
        lane128 = lane * TB

        def start_gather(p, b):
            pltpu.async_copy(table_hbm.at[idx_v.at[p]], gbuf[b], gsem[b])

        def wait_gather(p, b):
            pltpu.make_async_copy(
                table_hbm.at[idx_v.at[p]], gbuf[b], gsem[b]).wait()

        def out_off(p, tr):
            return ((p * (D_MODEL // 8) + tr) * NW + wid) * block_words

        def start_store(p, b):
            for tr in range(D_MODEL // 8):
                pltpu.async_copy(
                    tbuf[b].at[pl.ds(tr * block_words, block_words)],
                    out_hbm.at[pl.ds(out_off(p, tr), block_words)],
                    ssem[b])

        def wait_store(p, b):
            for tr in range(D_MODEL // 8):
                pltpu.make_async_copy(
                    tbuf[b].at[pl.ds(tr * block_words, block_words)],
                    out_hbm.at[pl.ds(out_off(p, tr), block_words)],
                    ssem[b]).wait()

        for b in range(NBUF):
            start_gather(b, b)

        def round_body(g, carry):
            for b in range(NBUF):
                p = g * NBUF + b
                wait_gather(p, b)

                @pl.when(g > 0)
                def _():
                    wait_store(p - NBUF, b)

                def tok_body(si, c):
                    for q in range(D_MODEL // 16):
                        v = gbuf[b][si, pl.ds(q * 16, 16)] * SCALE
                        idx = lane128 + (q * 16 * TB + si)
                        plsc.store_scatter(tbuf[b], [idx], v)
                    return c

                lax.fori_loop(0, TB, tok_body, 0, unroll=4)

                @pl.when(p + NBUF < P)
                def _():
                    start_gather(p + NBUF, b)

                start_store(p, b)
            return carry

        lax.fori_loop(0, n_rounds, round_body, 0)

        for b in range(NBUF):
            wait_store((n_rounds - 1) * NBUF + b, b)

    return emb


def kernel(x, table):
    S, P = x.shape
    xt = jnp.transpose(x.astype(jnp.int32))
    flat = _emb_call(S, P)(xt, table)
    out = flat.reshape(P, D_MODEL // 8, NW, 8, TB)
    return out.transpose(2, 4, 0, 1, 3).reshape(S, P, D_MODEL)
